# baseline (speedup 1.0000x reference)
import functools

import jax
import jax.numpy as jnp
from jax import lax
from jax.experimental import pallas as pl
from jax.experimental.pallas import tpu as pltpu
from jax.experimental.pallas import tpu_sc as plsc

B, L, DIM = 4096, 200, 64
LN = 16
NCH = (L + LN - 1) // LN


def _sc_pool(x, lengths, emb):
    info = plsc.get_sparse_core_info()
    nc, ns = info.num_cores, info.num_subcores
    nw = nc * ns
    bw = B // nw

    mesh = plsc.VectorSubcoreMesh(core_axis_name="c", subcore_axis_name="s")

    @functools.partial(
        pl.kernel,
        mesh=mesh,
        compiler_params=pltpu.CompilerParams(use_tc_tiling_on_sc=False),
        out_type=jax.ShapeDtypeStruct((B, DIM), jnp.float32),
        scratch_types=[
            pltpu.VMEM((bw, L), jnp.int32),
            pltpu.VMEM((bw + LN,), jnp.int32),
            pltpu.VMEM((NCH, LN), jnp.int32),
            pltpu.VMEM((NCH, LN, DIM), jnp.float32),
            pltpu.VMEM((bw, DIM), jnp.float32),
            pltpu.SemaphoreType.DMA,
        ],
    )
    def pool(x_hbm, len_hbm, emb_hbm, out_hbm, xb, lens, idx, gbuf, ob, sem):
        wid = lax.axis_index("s") * nc + lax.axis_index("c")
        base = wid * bw
        pltpu.sync_copy(x_hbm.at[pl.ds(base, bw), :], xb)
        pltpu.sync_copy(len_hbm.at[pl.ds(base, bw)], lens.at[pl.ds(0, bw)])

        def example(i, _):
            ln = lens[pl.ds(i, LN)][0]
            nch = (ln + (LN - 1)) // LN
            for j in range(NCH):
                off = j * LN if j < NCH - 1 else L - LN
                xv = xb[i, pl.ds(off, LN)]
                pos = lax.iota(jnp.int32, LN) + off
                valid = pos < ln
                if j == NCH - 1:
                    valid = valid & (pos >= (NCH - 1) * LN)
                idx[j, :] = jnp.where(valid, xv, 0)

            def fire(j, c):
                pltpu.make_async_copy(emb_hbm.at[idx.at[j]], gbuf.at[j], sem).start()
                return c

            lax.fori_loop(0, nch, fire, 0)

            def drain(j, acc):
                pltpu.make_async_copy(emb_hbm.at[idx.at[j]], gbuf.at[j], sem).wait()
                a0, a1, a2, a3 = acc
                for r in range(LN):
                    a0 = a0 + gbuf[j, r, pl.ds(0, LN)]
                    a1 = a1 + gbuf[j, r, pl.ds(LN, LN)]
                    a2 = a2 + gbuf[j, r, pl.ds(2 * LN, LN)]
                    a3 = a3 + gbuf[j, r, pl.ds(3 * LN, LN)]
                return (a0, a1, a2, a3)

            z = jnp.zeros((LN,), jnp.float32)
            acc = lax.fori_loop(0, nch, drain, (z, z, z, z))
            lnv = lax.broadcast_in_dim(ln.astype(jnp.float32), (LN,), ())
            for g in range(4):
                ob[i, pl.ds(g * LN, LN)] = acc[g] / lnv
            return 0

        lax.fori_loop(0, bw, example, 0)
        pltpu.sync_copy(ob, out_hbm.at[pl.ds(base, bw), :])

    return pool(x, lengths, emb)


def _mlp(avg, W1, b1, W2, b2):
    def body(a_ref, w1_ref, b1_ref, w2_ref, b2_ref, o_ref):
        a = a_ref[:, :]
        h = lax.dot_general(a, w1_ref[:, :], (((1,), (1,)), ((), ())),
                            preferred_element_type=jnp.float32)
        h = jnp.maximum(h + b1_ref[:][None, :], 0.0)
        lg = lax.dot_general(h, w2_ref[:, :], (((1,), (1,)), ((), ())),
                             preferred_element_type=jnp.float32)
        lg = lg + b2_ref[:][None, :]
        m = jnp.max(lg, axis=1, keepdims=True)
        s = jnp.log(jnp.sum(jnp.exp(lg - m), axis=1, keepdims=True)) + m
        o_ref[:, :] = lg - s

    return pl.pallas_call(
        body,
        out_shape=jax.ShapeDtypeStruct((B, W2.shape[0]), jnp.float32),
    )(avg, W1, b1, W2, b2)


def kernel(x, lengths, emb, W1, b1, W2, b2):
    x = x.astype(jnp.int32)
    lengths = lengths.astype(jnp.int32)
    avg = _sc_pool(x, lengths, emb)
    return _mlp(avg, W1, b1, W2, b2)
